# Initial kernel scaffold; baseline (speedup 1.0000x reference)
import functools

import jax
import jax.numpy as jnp
from jax import lax
from jax.experimental import pallas as pl
from jax.experimental.pallas import tpu as pltpu
from jax.experimental.pallas import tpu_sc as plsc

N = 10000
EFULL = 320000
D = 128
DE = 16

NC = 2
NS = 16
NW = NC * NS
CH = 128
NU_PER_W = 10112
NCHUNK = NU_PER_W // CH
EP = NU_PER_W * NW
YROWS = 10016
NPAD = 10240
ROWS_PER_SUB = NPAD // NS
F32 = jnp.float32


def _y_body(x_ref, w_ref, b_ref, o_ref):
    o_ref[...] = (
        jnp.dot(x_ref[...], w_ref[...], preferred_element_type=F32) + b_ref[...]
    )


def _compute_y(xp, wx, b2):
    return pl.pallas_call(
        _y_body,
        out_shape=jax.ShapeDtypeStruct((YROWS, D), F32),
    )(xp, wx, b2)


_EB = 2528


def _e_body(a_ref, w_ref, o_ref):
    o_ref[...] = jnp.dot(a_ref[...], w_ref[...], preferred_element_type=F32)


def _compute_e(ea_r, w8):
    return pl.pallas_call(
        _e_body,
        grid=(ea_r.shape[0] // _EB,),
        in_specs=[
            pl.BlockSpec((_EB, D), lambda i: (i, 0)),
            pl.BlockSpec((D, 8 * D), lambda i: (0, 0)),
        ],
        out_specs=pl.BlockSpec((_EB, 8 * D), lambda i: (i, 0)),
        out_shape=jax.ShapeDtypeStruct((ea_r.shape[0], 8 * D), F32),
    )(ea_r, w8)


def _sc_body(y_hbm, e_hbm, row_hbm, col_hbm, z128, z16,
             outm, outc, ybuf, ebuf, ones, ridx, cidx, accm, accc, sem):
    c = lax.axis_index("c")
    s = lax.axis_index("s")
    wid = s * NC + c

    r0 = s * ROWS_PER_SUB
    pltpu.sync_copy(z128.at[pl.ds(r0, ROWS_PER_SUB)],
                    accm.at[pl.ds(r0, ROWS_PER_SUB)])
    pltpu.sync_copy(z16.at[pl.ds(r0, ROWS_PER_SUB)],
                    accc.at[pl.ds(r0, ROWS_PER_SUB)])

    onev = jnp.full((16,), 1.0, dtype=F32)

    def _init_ones(r, carry):
        ones[r, :] = onev
        return carry

    lax.fori_loop(0, CH, _init_ones, 0)
    plsc.subcore_barrier()

    base_w = wid * NU_PER_W

    def _relu_add(r, carry):
        for cc in range(D // 16):
            sl = pl.ds(cc * 16, 16)
            ybuf[r, sl] = jnp.maximum(ybuf[r, sl] + ebuf[r, sl], 0.0)
        return carry

    def _chunk(k, carry):
        base = base_w + k * CH
        pltpu.sync_copy(row_hbm.at[pl.ds(base, CH)], ridx)
        pltpu.sync_copy(col_hbm.at[pl.ds(base, CH)], cidx)
        pltpu.sync_copy(e_hbm.at[pl.ds(base, CH)], ebuf)
        pltpu.async_copy(y_hbm.at[ridx], ybuf, sem).wait()
        lax.fori_loop(0, CH, _relu_add, 0)
        pltpu.sync_copy(ybuf, accm.at[cidx], add=True)
        pltpu.sync_copy(ones, accc.at[cidx], add=True)
        pltpu.async_copy(y_hbm.at[cidx], ybuf, sem).wait()
        lax.fori_loop(0, CH, _relu_add, 0)
        pltpu.sync_copy(ybuf, accm.at[ridx], add=True)
        pltpu.sync_copy(ones, accc.at[ridx], add=True)
        return carry

    lax.fori_loop(0, NCHUNK, _chunk, 0)
    plsc.subcore_barrier()

    for k in range(ROWS_PER_SUB // CH):
        rr = r0 + k * CH
        pltpu.sync_copy(accm.at[pl.ds(rr, CH)], ybuf)
        pltpu.sync_copy(ybuf, outm.at[c, pl.ds(rr, CH)])
        pltpu.sync_copy(accc.at[pl.ds(rr, CH)], ones)
        pltpu.sync_copy(ones, outc.at[c, pl.ds(rr, CH)])


_sc_call = functools.partial(
    pl.kernel,
    _sc_body,
    out_type=[
        jax.ShapeDtypeStruct((NC, NPAD, D), F32),
        jax.ShapeDtypeStruct((NC, NPAD, 16), F32),
    ],
    mesh=plsc.VectorSubcoreMesh(core_axis_name="c", subcore_axis_name="s"),
    scratch_types=[
        pltpu.VMEM((CH, D), F32),
        pltpu.VMEM((CH, D), F32),
        pltpu.VMEM((CH, 16), F32),
        pltpu.VMEM((CH,), jnp.int32),
        pltpu.VMEM((CH,), jnp.int32),
        pltpu.VMEM_SHARED((NPAD, D), F32),
        pltpu.VMEM_SHARED((NPAD, 16), F32),
        pltpu.SemaphoreType.DMA,
    ],
)


_CB = 1000


def _fin_body(pm_ref, pc_ref, x_ref, g_ref, b_ref, o_ref):
    pm = pm_ref[0] + pm_ref[1]
    cnt = pc_ref[0][:, 0:1] + pc_ref[1][:, 0:1]
    u = pm / jnp.maximum(cnt, 1.0) + x_ref[...]
    mu = jnp.mean(u, axis=1, keepdims=True)
    d = u - mu
    var = jnp.mean(d * d, axis=1, keepdims=True)
    o_ref[...] = d * lax.rsqrt(var + 1e-5) * g_ref[...] + b_ref[...]


def _finalize(pm, pc, x, g2, be2):
    return pl.pallas_call(
        _fin_body,
        grid=(N // _CB,),
        in_specs=[
            pl.BlockSpec((NC, _CB, D), lambda i: (0, i, 0)),
            pl.BlockSpec((NC, _CB, 16), lambda i: (0, i, 0)),
            pl.BlockSpec((_CB, D), lambda i: (i, 0)),
            pl.BlockSpec((1, D), lambda i: (0, 0)),
            pl.BlockSpec((1, D), lambda i: (0, 0)),
        ],
        out_specs=pl.BlockSpec((_CB, D), lambda i: (i, 0)),
        out_shape=jax.ShapeDtypeStruct((N, D), F32),
    )(pm, pc, x, g2, be2)


def kernel(x, edge_index, edge_attr, W_msg, b_msg, ln_gamma, ln_beta):
    row = edge_index[0]
    col = edge_index[1]
    pad = EP - EFULL
    rowp = jnp.concatenate([row, jnp.full((pad,), N, dtype=jnp.int32)])
    colp = jnp.concatenate([col, jnp.full((pad,), N, dtype=jnp.int32)])
    eap = jnp.concatenate([edge_attr, jnp.zeros((pad, DE), dtype=F32)])
    ea_r = eap.reshape(EP // 8, 8 * DE)
    w8 = jnp.kron(jnp.eye(8, dtype=F32), W_msg[D:])
    xp = jnp.concatenate([x, jnp.zeros((YROWS - N, D), dtype=F32)])
    b2 = b_msg.reshape(1, D)

    y = _compute_y(xp, W_msg[:D], b2)
    e = _compute_e(ea_r, w8).reshape(EP, D)

    z128 = jnp.zeros((NPAD, D), dtype=F32)
    z16 = jnp.zeros((NPAD, 16), dtype=F32)
    pm, pc = _sc_call()(y, e, rowp, colp, z128, z16)

    return _finalize(pm, pc, x, ln_gamma.reshape(1, D), ln_beta.reshape(1, D))


# speedup vs baseline: 3.5846x; 3.5846x over previous
import functools

import jax
import jax.numpy as jnp
from jax import lax
from jax.experimental import pallas as pl
from jax.experimental.pallas import tpu as pltpu
from jax.experimental.pallas import tpu_sc as plsc

N = 10000
EFULL = 320000
D = 128
DE = 16

NC = 2
NS = 16
NW = NC * NS
CH = 128
NU_PER_W = 10112
NCHUNK = NU_PER_W // CH
EP = NU_PER_W * NW
YROWS = 10016
NPAD = 10112
ROWS_PER_SUB = NPAD // NS
F32 = jnp.float32


def _y_body(x_ref, w_ref, b_ref, o_ref):
    o_ref[...] = (
        jnp.dot(x_ref[...], w_ref[...], preferred_element_type=F32) + b_ref[...]
    )


def _compute_y(xp, wx, b2):
    return pl.pallas_call(
        _y_body,
        out_shape=jax.ShapeDtypeStruct((YROWS, D), F32),
    )(xp, wx, b2)


_EB = 2528


def _e_body(a_ref, w_ref, o_ref):
    o_ref[...] = jnp.dot(a_ref[...], w_ref[...], preferred_element_type=F32)


def _compute_e(ea_r, w8):
    return pl.pallas_call(
        _e_body,
        grid=(ea_r.shape[0] // _EB,),
        in_specs=[
            pl.BlockSpec((_EB, D), lambda i: (i, 0)),
            pl.BlockSpec((D, 8 * D), lambda i: (0, 0)),
        ],
        out_specs=pl.BlockSpec((_EB, 8 * D), lambda i: (i, 0)),
        out_shape=jax.ShapeDtypeStruct((ea_r.shape[0], 8 * D), F32),
    )(ea_r, w8)


def _sc_body(y_hbm, e_hbm, row_hbm, col_hbm, z128, z1,
             outm, outc, ybuf, ebuf, ridx, cidx, cnt, accm, sem):
    c = lax.axis_index("c")
    s = lax.axis_index("s")
    wid = s * NC + c

    r0 = s * ROWS_PER_SUB
    pltpu.sync_copy(z128.at[pl.ds(r0, ROWS_PER_SUB)],
                    accm.at[pl.ds(r0, ROWS_PER_SUB)])
    pltpu.sync_copy(z1, cnt)
    plsc.subcore_barrier()

    base_w = wid * NU_PER_W

    def _relu_add(r, carry):
        for cc in range(D // 16):
            sl = pl.ds(cc * 16, 16)
            ybuf[r, sl] = jnp.maximum(ybuf[r, sl] + ebuf[r, sl], 0.0)
        return carry

    onehot = jnp.maximum(1 - lax.iota(jnp.int32, 16), 0).astype(F32)

    def _count(dst_ref):
        def body(g, carry):
            v16 = dst_ref[pl.ds(g * 16, 16)]
            for lane in range(16):
                i = v16[lane]
                cnt[pl.ds(i, 16)] = cnt[pl.ds(i, 16)] + onehot
            return carry
        return body

    def _chunk(k, carry):
        base = base_w + k * CH
        pltpu.sync_copy(row_hbm.at[pl.ds(base, CH)], ridx)
        pltpu.sync_copy(col_hbm.at[pl.ds(base, CH)], cidx)
        pltpu.sync_copy(e_hbm.at[pl.ds(base, CH)], ebuf)
        pltpu.async_copy(y_hbm.at[ridx], ybuf, sem).wait()
        lax.fori_loop(0, CH, _relu_add, 0)
        pltpu.sync_copy(ybuf, accm.at[cidx], add=True)
        lax.fori_loop(0, CH // 16, _count(cidx), 0)
        pltpu.async_copy(y_hbm.at[cidx], ybuf, sem).wait()
        lax.fori_loop(0, CH, _relu_add, 0)
        pltpu.sync_copy(ybuf, accm.at[ridx], add=True)
        lax.fori_loop(0, CH // 16, _count(ridx), 0)
        return carry

    lax.fori_loop(0, NCHUNK, _chunk, 0)
    plsc.subcore_barrier()

    off = 0
    for sz in (CH, CH, CH, CH, ROWS_PER_SUB - 4 * CH):
        rr = r0 + off
        pltpu.sync_copy(accm.at[pl.ds(rr, sz)], ybuf.at[pl.ds(0, sz)])
        pltpu.sync_copy(ybuf.at[pl.ds(0, sz)], outm.at[c, pl.ds(rr, sz)])
        off += sz
    pltpu.sync_copy(cnt, outc.at[c, s])


@functools.cache
def _sc_call():
  return pl.kernel(
    _sc_body,
    out_type=[
        jax.ShapeDtypeStruct((NC, NPAD, D), F32),
        jax.ShapeDtypeStruct((NC, NS, NPAD), F32),
    ],
    mesh=plsc.VectorSubcoreMesh(
        core_axis_name="c", subcore_axis_name="s",
        num_cores=NC, num_subcores=NS),
    scratch_types=[
        pltpu.VMEM((CH, D), F32),
        pltpu.VMEM((CH, D), F32),
        pltpu.VMEM((CH,), jnp.int32),
        pltpu.VMEM((CH,), jnp.int32),
        pltpu.VMEM((NPAD,), F32),
        pltpu.VMEM_SHARED((NPAD, D), F32),
        pltpu.SemaphoreType.DMA,
    ],
)


def _fin_body(pm_ref, pc_ref, x_ref, g_ref, b_ref, o_ref):
    pm = pm_ref[0] + pm_ref[1]
    cnt = jnp.sum(pc_ref[...], axis=0)[:, None]
    u = pm / jnp.maximum(cnt, 1.0) + x_ref[...]
    mu = jnp.mean(u, axis=1, keepdims=True)
    d = u - mu
    var = jnp.mean(d * d, axis=1, keepdims=True)
    o_ref[...] = d * lax.rsqrt(var + 1e-5) * g_ref[...] + b_ref[...]


def _finalize(pm, pc, xp2, g2, be2):
    return pl.pallas_call(
        _fin_body,
        out_shape=jax.ShapeDtypeStruct((NPAD, D), F32),
    )(pm, pc, xp2, g2, be2)


def kernel(x, edge_index, edge_attr, W_msg, b_msg, ln_gamma, ln_beta):
    row = edge_index[0]
    col = edge_index[1]
    pad = EP - EFULL
    rowp = jnp.concatenate([row, jnp.full((pad,), N, dtype=jnp.int32)])
    colp = jnp.concatenate([col, jnp.full((pad,), N, dtype=jnp.int32)])
    eap = jnp.concatenate([edge_attr, jnp.zeros((pad, DE), dtype=F32)])
    ea_r = eap.reshape(EP // 8, 8 * DE)
    w8 = jnp.kron(jnp.eye(8, dtype=F32), W_msg[D:])
    xp = jnp.concatenate([x, jnp.zeros((YROWS - N, D), dtype=F32)])
    b2 = b_msg.reshape(1, D)

    y = _compute_y(xp, W_msg[:D], b2)
    e = _compute_e(ea_r, w8).reshape(EP, D)

    z128 = jnp.zeros((NPAD, D), dtype=F32)
    z1 = jnp.zeros((NPAD,), dtype=F32)
    pm, pc = _sc_call()(y, e, rowp, colp, z128, z1)

    xp2 = jnp.concatenate([x, jnp.zeros((NPAD - N, D), dtype=F32)])
    out = _finalize(pm, pc.reshape(NC * NS, NPAD), xp2,
                    ln_gamma.reshape(1, D), ln_beta.reshape(1, D))
    return out[:N]
